# baseline (speedup 1.0000x reference)
import functools

import jax
import jax.numpy as jnp
from jax import lax
from jax.experimental import pallas as pl
from jax.experimental.pallas import tpu as pltpu
from jax.experimental.pallas import tpu_sc as plsc

N = 10000
E = 320000
D_IN = 128
D_HID = 16
D_OUT = 64

NC = 2
NS = 16
NW = NC * NS
EPT = E // NW
B = 128
NB = 79
EPTP = NB * B
N_PAD = 10240
RPT = N_PAD // NS

_MESH = dict(core_axis_name="c", subcore_axis_name="s", num_cores=NC,
             num_subcores=NS)

_SC_CACHE = {}

_SC_PARAMS = pltpu.CompilerParams(use_tc_tiling_on_sc=False)


def _make_edge_pass(d):
  @functools.partial(
      pl.kernel,
      out_type=jax.ShapeDtypeStruct((NC, N_PAD, d), jnp.float32),
      mesh=plsc.VectorSubcoreMesh(**_MESH),
      scratch_types=[
          pltpu.VMEM((NB, B), jnp.int32),
          pltpu.VMEM((NB, B), jnp.int32),
          pltpu.VMEM((B, d), jnp.float32),
          pltpu.VMEM((B, d), jnp.float32),
          pltpu.SemaphoreType.DMA,
          pltpu.SemaphoreType.DMA,
          pltpu.VMEM_SHARED((N_PAD, d), jnp.float32),
      ],
      compiler_params=_SC_PARAMS,
  )
  def edge_pass(y_hbm, src_hbm, dst_hbm, out_hbm, srcv, dstv, buf0,
                buf1, sem0, sem1, acc):
    cid = lax.axis_index("c")
    sid = lax.axis_index("s")
    wid = sid * NC + cid
    pltpu.sync_copy(src_hbm.at[wid], srcv)
    pltpu.sync_copy(dst_hbm.at[wid], dstv)

    def zrow(i, c):
      for cc in range(d // 16):
        buf0[i, pl.ds(cc * 16, 16)] = jnp.zeros((16,), jnp.float32)
      return c

    lax.fori_loop(0, B, zrow, 0, unroll=False)
    for r in range(RPT // B):
      pltpu.sync_copy(buf0, acc.at[pl.ds(sid * RPT + r * B, B)])
    plsc.subcore_barrier()

    bufs = (buf0, buf1)
    sems = (sem0, sem1)

    def gather(j, b):
      pltpu.async_copy(y_hbm.at[srcv.at[j]], bufs[b], sems[b])

    def gwait(j, b):
      pltpu.make_async_copy(y_hbm.at[srcv.at[j]], bufs[b], sems[b]).wait()

    def scat(j, b):
      pltpu.sync_copy(bufs[b], acc.at[dstv.at[j]], add=True)

    gather(0, 0)

    def step(i, carry):
      j0 = 2 * i
      gather(j0 + 1, 1)
      gwait(j0, 0)
      scat(j0, 0)
      gather(j0 + 2, 0)
      gwait(j0 + 1, 1)
      scat(j0 + 1, 1)
      return carry

    lax.fori_loop(0, (NB - 1) // 2, step, 0, unroll=False)
    gwait(NB - 1, 0)
    scat(NB - 1, 0)
    plsc.subcore_barrier()
    pltpu.sync_copy(acc.at[pl.ds(sid * RPT, RPT)],
                    out_hbm.at[cid, pl.ds(sid * RPT, RPT)])

  return edge_pass


def _make_deg_pass():
  @functools.partial(
      pl.kernel,
      out_type=jax.ShapeDtypeStruct((NC, N_PAD, D_HID), jnp.float32),
      mesh=plsc.VectorSubcoreMesh(**_MESH),
      scratch_types=[
          pltpu.VMEM((NB, B), jnp.int32),
          pltpu.VMEM((B, D_HID), jnp.float32),
          pltpu.SemaphoreType.DMA,
          pltpu.VMEM_SHARED((N_PAD, D_HID), jnp.float32),
      ],
      compiler_params=_SC_PARAMS,
  )
  def deg_pass(dst_hbm, out_hbm, dstv, buf, sem, acc):
    cid = lax.axis_index("c")
    sid = lax.axis_index("s")
    wid = sid * NC + cid
    pltpu.sync_copy(dst_hbm.at[wid], dstv)

    def fill(val):
      def frow(i, c):
        buf[i, :] = jnp.full((16,), val, jnp.float32)
        return c
      lax.fori_loop(0, B, frow, 0, unroll=False)

    fill(0.0)
    for r in range(RPT // B):
      pltpu.sync_copy(buf, acc.at[pl.ds(sid * RPT + r * B, B)])
    fill(1.0)
    plsc.subcore_barrier()

    def fire(j, c):
      pltpu.async_copy(buf, acc.at[dstv.at[j]], sem, add=True)
      return c

    def drain(j, c):
      pltpu.make_async_copy(buf, acc.at[dstv.at[j]], sem).wait()
      return c

    lax.fori_loop(0, NB, fire, 0, unroll=False)
    lax.fori_loop(0, NB, drain, 0, unroll=False)
    plsc.subcore_barrier()
    pltpu.sync_copy(acc.at[pl.ds(sid * RPT, RPT)],
                    out_hbm.at[cid, pl.ds(sid * RPT, RPT)])

  return deg_pass


def _sc_kernels():
  if not _SC_CACHE:
    _SC_CACHE["edge16"] = _make_edge_pass(D_HID)
    _SC_CACHE["edge64"] = _make_edge_pass(D_OUT)
    _SC_CACHE["deg"] = _make_deg_pass()
  return _SC_CACHE["deg"], _SC_CACHE["edge16"], _SC_CACHE["edge64"]


_BM = 2000


def _dinv_of(degp):
  return lax.rsqrt(degp[0, :, 0:1] + degp[1, :, 0:1] + 1.0)


def _tc0_body(x, w1, ho):
  ho[...] = jnp.dot(x[...], w1[...], preferred_element_type=jnp.float32)


def _tc1_body(dg, h, yo):
  yo[...] = h[...] * _dinv_of(dg[...])


def _tc2_body(dg, p, y, w2, b1, zo):
  di = _dinv_of(dg[...])
  h = jnp.maximum((p[0] + p[1] + y[...]) * di + b1[...], 0.0)
  zo[...] = jnp.dot(h, w2[...], preferred_element_type=jnp.float32) * di


def _tc3_body(dg, p, z, b2, o):
  a = (p[0] + p[1] + z[...]) * _dinv_of(dg[...]) + b2[...]
  m = jnp.max(a, axis=1, keepdims=True)
  ex = jnp.exp(a - m)
  o[...] = a - (jnp.log(jnp.sum(ex, axis=1, keepdims=True)) + m)


def _row_spec(d):
  return pl.BlockSpec((_BM, d), lambda m: (m, 0))


def _part_spec(d):
  return pl.BlockSpec((NC, _BM, d), lambda m: (0, m, 0))


def _full_spec(r, d):
  return pl.BlockSpec((r, d), lambda m: (0, 0))


_GRID = N // _BM

_tc0 = pl.pallas_call(
    _tc0_body,
    grid=(_GRID,),
    in_specs=[_row_spec(D_IN), _full_spec(D_IN, D_HID)],
    out_specs=_row_spec(D_HID),
    out_shape=jax.ShapeDtypeStruct((N_PAD, D_HID), jnp.float32),
)

_tc1 = pl.pallas_call(
    _tc1_body,
    grid=(_GRID,),
    in_specs=[_part_spec(D_HID), _row_spec(D_HID)],
    out_specs=_row_spec(D_HID),
    out_shape=jax.ShapeDtypeStruct((N_PAD, D_HID), jnp.float32),
)

_tc2 = pl.pallas_call(
    _tc2_body,
    grid=(_GRID,),
    in_specs=[_part_spec(D_HID), _part_spec(D_HID), _row_spec(D_HID),
              _full_spec(D_HID, D_OUT), _full_spec(1, D_HID)],
    out_specs=_row_spec(D_OUT),
    out_shape=jax.ShapeDtypeStruct((N_PAD, D_OUT), jnp.float32),
)

_tc3 = pl.pallas_call(
    _tc3_body,
    grid=(_GRID,),
    in_specs=[_part_spec(D_HID), _part_spec(D_OUT), _row_spec(D_OUT),
              _full_spec(1, D_OUT)],
    out_specs=_row_spec(D_OUT),
    out_shape=jax.ShapeDtypeStruct((N, D_OUT), jnp.float32),
)


def kernel(x, edge_index, W1, b1, W2, b2):
  ei = edge_index.astype(jnp.int32)
  pad = jnp.full((NW, EPTP - EPT), N, jnp.int32)
  src3 = jnp.concatenate([ei[0].reshape(NW, EPT), pad], 1).reshape(NW, NB, B)
  dst3 = jnp.concatenate([ei[1].reshape(NW, EPT), pad], 1).reshape(NW, NB, B)

  _deg, _edge16, _edge64 = _sc_kernels()
  degp = _deg(dst3)
  h1 = _tc0(x, W1)
  y1 = _tc1(degp, h1)
  p1 = _edge16(y1, src3, dst3)
  z = _tc2(degp, p1, y1, W2, b1.reshape(1, D_HID))
  p2 = _edge64(z, src3, dst3)
  return _tc3(degp, p2, z, b2.reshape(1, D_OUT))


# speedup vs baseline: 1.2127x; 1.2127x over previous
import functools

import jax
import jax.numpy as jnp
from jax import lax
from jax.experimental import pallas as pl
from jax.experimental.pallas import tpu as pltpu
from jax.experimental.pallas import tpu_sc as plsc

N = 10000
E = 320000
D_IN = 128
D_HID = 16
D_OUT = 64

NC = 2
NS = 16
NW = NC * NS
EPT = E // NW
B = 80
NB = EPT // B
N_PAD = 10240
RPT = N_PAD // NS

_MESH = dict(core_axis_name="c", subcore_axis_name="s", num_cores=NC,
             num_subcores=NS)

_SC_CACHE = {}

_SC_PARAMS = pltpu.CompilerParams(use_tc_tiling_on_sc=False)


def _make_edge_pass(d):
  @functools.partial(
      pl.kernel,
      out_type=jax.ShapeDtypeStruct((NC, N_PAD, d), jnp.float32),
      mesh=plsc.VectorSubcoreMesh(**_MESH),
      scratch_types=[
          pltpu.VMEM((NB, B), jnp.int32),
          pltpu.VMEM((NB, B), jnp.int32),
          pltpu.VMEM((B, d), jnp.float32),
          pltpu.VMEM((B, d), jnp.float32),
          pltpu.SemaphoreType.DMA,
          pltpu.SemaphoreType.DMA,
          pltpu.VMEM_SHARED((N_PAD, d), jnp.float32),
      ],
      compiler_params=_SC_PARAMS,
  )
  def edge_pass(y_hbm, src_hbm, dst_hbm, out_hbm, srcv, dstv, buf0,
                buf1, sem0, sem1, acc):
    cid = lax.axis_index("c")
    sid = lax.axis_index("s")
    wid = sid * NC + cid
    pltpu.sync_copy(src_hbm.at[wid], srcv)
    pltpu.sync_copy(dst_hbm.at[wid], dstv)

    def zrow(i, c):
      for cc in range(d // 16):
        buf0[i, pl.ds(cc * 16, 16)] = jnp.zeros((16,), jnp.float32)
      return c

    lax.fori_loop(0, B, zrow, 0, unroll=False)
    for r in range(RPT // B):
      pltpu.sync_copy(buf0, acc.at[pl.ds(sid * RPT + r * B, B)])
    plsc.subcore_barrier()

    bufs = (buf0, buf1)
    sems = (sem0, sem1)

    def gather(j, b):
      pltpu.async_copy(y_hbm.at[srcv.at[j]], bufs[b], sems[b])

    def gwait(j, b):
      pltpu.make_async_copy(y_hbm.at[srcv.at[j]], bufs[b], sems[b]).wait()

    def scat(j, b):
      pltpu.sync_copy(bufs[b], acc.at[dstv.at[j]], add=True)

    gather(0, 0)

    def step(i, carry):
      j0 = 2 * i
      gather(j0 + 1, 1)
      gwait(j0, 0)
      scat(j0, 0)
      gather(j0 + 2, 0)
      gwait(j0 + 1, 1)
      scat(j0 + 1, 1)
      return carry

    lax.fori_loop(0, (NB - 1) // 2, step, 0, unroll=False)
    gwait(NB - 1, 0)
    scat(NB - 1, 0)
    plsc.subcore_barrier()
    pltpu.sync_copy(acc.at[pl.ds(sid * RPT, RPT)],
                    out_hbm.at[cid, pl.ds(sid * RPT, RPT)])

  return edge_pass


def _make_deg_pass():
  @functools.partial(
      pl.kernel,
      out_type=jax.ShapeDtypeStruct((NC, N_PAD, D_HID), jnp.float32),
      mesh=plsc.VectorSubcoreMesh(**_MESH),
      scratch_types=[
          pltpu.VMEM((NB, B), jnp.int32),
          pltpu.VMEM((B, D_HID), jnp.float32),
          pltpu.SemaphoreType.DMA,
          pltpu.VMEM_SHARED((N_PAD, D_HID), jnp.float32),
      ],
      compiler_params=_SC_PARAMS,
  )
  def deg_pass(dst_hbm, out_hbm, dstv, buf, sem, acc):
    cid = lax.axis_index("c")
    sid = lax.axis_index("s")
    wid = sid * NC + cid
    pltpu.sync_copy(dst_hbm.at[wid], dstv)

    def fill(val):
      def frow(i, c):
        buf[i, :] = jnp.full((16,), val, jnp.float32)
        return c
      lax.fori_loop(0, B, frow, 0, unroll=False)

    fill(0.0)
    for r in range(RPT // B):
      pltpu.sync_copy(buf, acc.at[pl.ds(sid * RPT + r * B, B)])
    fill(1.0)
    plsc.subcore_barrier()

    def fire(j, c):
      pltpu.async_copy(buf, acc.at[dstv.at[j]], sem, add=True)
      return c

    def drain(j, c):
      pltpu.make_async_copy(buf, acc.at[dstv.at[j]], sem).wait()
      return c

    lax.fori_loop(0, NB, fire, 0, unroll=False)
    lax.fori_loop(0, NB, drain, 0, unroll=False)
    plsc.subcore_barrier()
    pltpu.sync_copy(acc.at[pl.ds(sid * RPT, RPT)],
                    out_hbm.at[cid, pl.ds(sid * RPT, RPT)])

  return deg_pass


def _sc_kernels():
  if not _SC_CACHE:
    _SC_CACHE["edge16"] = _make_edge_pass(D_HID)
    _SC_CACHE["edge64"] = _make_edge_pass(D_OUT)
    _SC_CACHE["deg"] = _make_deg_pass()
  return _SC_CACHE["deg"], _SC_CACHE["edge16"], _SC_CACHE["edge64"]


_BM = 2000


def _dinv_of(degp):
  return lax.rsqrt(degp[0, :, 0:1] + degp[1, :, 0:1] + 1.0)


def _tc0_body(x, w1, ho):
  ho[...] = jnp.dot(x[...], w1[...], preferred_element_type=jnp.float32)


def _tc1_body(dg, h, yo):
  yo[...] = h[...] * _dinv_of(dg[...])


def _tc2_body(dg, p, y, w2, b1, zo):
  di = _dinv_of(dg[...])
  h = jnp.maximum((p[0] + p[1] + y[...]) * di + b1[...], 0.0)
  zo[...] = jnp.dot(h, w2[...], preferred_element_type=jnp.float32) * di


def _tc3_body(dg, p, z, b2, o):
  a = (p[0] + p[1] + z[...]) * _dinv_of(dg[...]) + b2[...]
  m = jnp.max(a, axis=1, keepdims=True)
  ex = jnp.exp(a - m)
  o[...] = a - (jnp.log(jnp.sum(ex, axis=1, keepdims=True)) + m)


def _row_spec(d):
  return pl.BlockSpec((_BM, d), lambda m: (m, 0))


def _part_spec(d):
  return pl.BlockSpec((NC, _BM, d), lambda m: (0, m, 0))


def _full_spec(r, d):
  return pl.BlockSpec((r, d), lambda m: (0, 0))


_GRID = N // _BM

_tc0 = pl.pallas_call(
    _tc0_body,
    grid=(_GRID,),
    in_specs=[_row_spec(D_IN), _full_spec(D_IN, D_HID)],
    out_specs=_row_spec(D_HID),
    out_shape=jax.ShapeDtypeStruct((N, D_HID), jnp.float32),
)

_tc1 = pl.pallas_call(
    _tc1_body,
    grid=(_GRID,),
    in_specs=[_part_spec(D_HID), _row_spec(D_HID)],
    out_specs=_row_spec(D_HID),
    out_shape=jax.ShapeDtypeStruct((N, D_HID), jnp.float32),
)

_tc2 = pl.pallas_call(
    _tc2_body,
    grid=(_GRID,),
    in_specs=[_part_spec(D_HID), _part_spec(D_HID), _row_spec(D_HID),
              _full_spec(D_HID, D_OUT), _full_spec(1, D_HID)],
    out_specs=_row_spec(D_OUT),
    out_shape=jax.ShapeDtypeStruct((N, D_OUT), jnp.float32),
)

_tc3 = pl.pallas_call(
    _tc3_body,
    grid=(_GRID,),
    in_specs=[_part_spec(D_HID), _part_spec(D_OUT), _row_spec(D_OUT),
              _full_spec(1, D_OUT)],
    out_specs=_row_spec(D_OUT),
    out_shape=jax.ShapeDtypeStruct((N, D_OUT), jnp.float32),
)


def kernel(x, edge_index, W1, b1, W2, b2):
  ei = edge_index.astype(jnp.int32)
  src3 = ei[0].reshape(NW, NB, B)
  dst3 = ei[1].reshape(NW, NB, B)

  _deg, _edge16, _edge64 = _sc_kernels()
  degp = _deg(dst3)
  h1 = _tc0(x, W1)
  y1 = _tc1(degp, h1)
  p1 = _edge16(y1, src3, dst3)
  z = _tc2(degp, p1, y1, W2, b1.reshape(1, D_HID))
  p2 = _edge64(z, src3, dst3)
  return _tc3(degp, p2, z, b2.reshape(1, D_OUT))


# speedup vs baseline: 1.6746x; 1.3810x over previous
import functools

import jax
import jax.numpy as jnp
from jax import lax
from jax.experimental import pallas as pl
from jax.experimental.pallas import tpu as pltpu
from jax.experimental.pallas import tpu_sc as plsc

N = 10000
E = 320000
D_IN = 128
D_HID = 16
D_OUT = 64

NC = 2
NS = 16
NW = NC * NS
EPT = E // NW
B = 80
NB = EPT // B
N_PAD = 10240
RPT = N_PAD // NS

_MESH = dict(core_axis_name="c", subcore_axis_name="s", num_cores=NC,
             num_subcores=NS)

_SC_CACHE = {}

_SC_PARAMS = pltpu.CompilerParams(use_tc_tiling_on_sc=False)


def _make_edge_pass(d):
  @functools.partial(
      pl.kernel,
      out_type=jax.ShapeDtypeStruct((NC, N_PAD, d), jnp.float32),
      mesh=plsc.VectorSubcoreMesh(**_MESH),
      scratch_types=[
          pltpu.VMEM((NB, B), jnp.int32),
          pltpu.VMEM((NB, B), jnp.int32),
          pltpu.VMEM((B, d), jnp.float32),
          pltpu.VMEM((B, d), jnp.float32),
          pltpu.SemaphoreType.DMA,
          pltpu.SemaphoreType.DMA,
          pltpu.VMEM_SHARED((N_PAD, d), jnp.float32),
      ],
      compiler_params=_SC_PARAMS,
  )
  def edge_pass(y_hbm, src_hbm, dst_hbm, out_hbm, srcv, dstv, buf0,
                buf1, sem0, sem1, acc):
    cid = lax.axis_index("c")
    sid = lax.axis_index("s")
    wid = sid * NC + cid
    pltpu.sync_copy(src_hbm.at[wid], srcv)
    pltpu.sync_copy(dst_hbm.at[wid], dstv)

    def zrow(i, c):
      for cc in range(d // 16):
        buf0[i, pl.ds(cc * 16, 16)] = jnp.zeros((16,), jnp.float32)
      return c

    lax.fori_loop(0, B, zrow, 0, unroll=False)
    for r in range(RPT // B):
      pltpu.sync_copy(buf0, acc.at[pl.ds(sid * RPT + r * B, B)])
    plsc.subcore_barrier()

    bufs = (buf0, buf1)
    sems = (sem0, sem1)

    def gather(j, b):
      pltpu.async_copy(y_hbm.at[srcv.at[j]], bufs[b], sems[b])

    def gwait(j, b):
      pltpu.make_async_copy(y_hbm.at[srcv.at[j]], bufs[b], sems[b]).wait()

    def scat(j, b):
      pltpu.sync_copy(bufs[b], acc.at[dstv.at[j]], add=True)

    gather(0, 0)

    def step(i, carry):
      j0 = 2 * i
      gather(j0 + 1, 1)
      gwait(j0, 0)
      scat(j0, 0)
      gather(j0 + 2, 0)
      gwait(j0 + 1, 1)
      scat(j0 + 1, 1)
      return carry

    lax.fori_loop(0, (NB - 1) // 2, step, 0, unroll=False)
    gwait(NB - 1, 0)
    scat(NB - 1, 0)
    plsc.subcore_barrier()
    pltpu.sync_copy(acc.at[pl.ds(sid * RPT, RPT)],
                    out_hbm.at[cid, pl.ds(sid * RPT, RPT)])

  return edge_pass


def _make_deg_pass():
  @functools.partial(
      pl.kernel,
      out_type=jax.ShapeDtypeStruct((NC, N_PAD, D_HID), jnp.float32),
      mesh=plsc.VectorSubcoreMesh(**_MESH),
      scratch_types=[
          pltpu.VMEM((NB, B), jnp.int32),
          pltpu.VMEM((B, D_HID), jnp.float32),
          pltpu.SemaphoreType.DMA,
          pltpu.VMEM_SHARED((N_PAD, D_HID), jnp.float32),
      ],
      compiler_params=_SC_PARAMS,
  )
  def deg_pass(dst_hbm, out_hbm, dstv, buf, sem, acc):
    cid = lax.axis_index("c")
    sid = lax.axis_index("s")
    wid = sid * NC + cid
    pltpu.sync_copy(dst_hbm.at[wid], dstv)

    def fill(val):
      def frow(i, c):
        buf[i, :] = jnp.full((16,), val, jnp.float32)
        return c
      lax.fori_loop(0, B, frow, 0, unroll=False)

    fill(0.0)
    for r in range(RPT // B):
      pltpu.sync_copy(buf, acc.at[pl.ds(sid * RPT + r * B, B)])
    fill(1.0)
    plsc.subcore_barrier()

    def fire(j, c):
      pltpu.async_copy(buf, acc.at[dstv.at[j]], sem, add=True)
      return c

    def drain(j, c):
      pltpu.make_async_copy(buf, acc.at[dstv.at[j]], sem).wait()
      return c

    lax.fori_loop(0, NB, fire, 0, unroll=False)
    lax.fori_loop(0, NB, drain, 0, unroll=False)
    plsc.subcore_barrier()
    pltpu.sync_copy(acc.at[pl.ds(sid * RPT, RPT)],
                    out_hbm.at[cid, pl.ds(sid * RPT, RPT)])

  return deg_pass


def _sc_kernels():
  if not _SC_CACHE:
    _SC_CACHE["edge16"] = _make_edge_pass(D_HID)
    _SC_CACHE["deg"] = _make_deg_pass()
  return _SC_CACHE["deg"], _SC_CACHE["edge16"]


_NF16 = N * D_HID // 128


def _tca_body(x8, w1b, dg, yo):
  deg = dg[0, :_NF16] + dg[1, :_NF16] + 1.0
  h = jnp.dot(x8[...], w1b[...], preferred_element_type=jnp.float32)
  yo[...] = h * lax.rsqrt(deg)


def _tcb_body(dg, p1f, y1f, b1f, ho):
  dinv = lax.rsqrt(dg[0, :_NF16] + dg[1, :_NF16] + 1.0)
  h = jnp.maximum(
      (p1f[0, :_NF16] + p1f[1, :_NF16] + y1f[...]) * dinv + b1f[...], 0.0)
  ho[...] = h * dinv


def _tcc_body(dg, p2f, hdf, w2, b2, o):
  dinv = lax.rsqrt(dg[0, :_NF16] + dg[1, :_NF16] + 1.0)
  f = (p2f[0, :_NF16] + p2f[1, :_NF16] + hdf[...]) * dinv
  for c in range(8):
    fc = f[:, c * 16:(c + 1) * 16]
    a = jnp.dot(fc, w2[...], preferred_element_type=jnp.float32) + b2[...]
    m = jnp.max(a, axis=1, keepdims=True)
    ex = jnp.exp(a - m)
    o[:, c, :] = a - (jnp.log(jnp.sum(ex, axis=1, keepdims=True)) + m)


_tca = pl.pallas_call(
    _tca_body,
    out_shape=jax.ShapeDtypeStruct((_NF16, 128), jnp.float32),
)

_tcb = pl.pallas_call(
    _tcb_body,
    out_shape=jax.ShapeDtypeStruct((_NF16, 128), jnp.float32),
)

_tcc = pl.pallas_call(
    _tcc_body,
    out_shape=jax.ShapeDtypeStruct((_NF16, 8, D_OUT), jnp.float32),
)


def kernel(x, edge_index, W1, b1, W2, b2):
  ei = edge_index.astype(jnp.int32)
  src3 = ei[0].reshape(NW, NB, B)
  dst3 = ei[1].reshape(NW, NB, B)

  _deg, _edge16 = _sc_kernels()
  degp = _deg(dst3)
  degf = degp.reshape(NC, 1280, 128)

  x8 = x.reshape(1250, 1024)
  w1b = jnp.kron(jnp.eye(8, dtype=jnp.float32), W1)
  y1f = _tca(x8, w1b, degf)

  p1 = _edge16(y1f.reshape(N, D_HID), src3, dst3)
  p1f = p1.reshape(NC, 1280, 128)
  b1f = jnp.tile(b1, 8).reshape(1, 128)
  hdf = _tcb(degf, p1f, y1f, b1f)

  p2 = _edge16(hdf.reshape(N, D_HID), src3, dst3)
  p2f = p2.reshape(NC, 1280, 128)
  out3 = _tcc(degf, p2f, hdf, W2, b2.reshape(1, D_OUT))
  return out3.reshape(N, D_OUT)
